# Initial kernel scaffold; baseline (speedup 1.0000x reference)
import jax
import jax.numpy as jnp
from jax.experimental import pallas as pl


def kernel(user_idx, pos_item_idx, neg_item_idx, user_tables, item_table, W1, b1, W2, b2, W3, b3):
    raise NotImplementedError("write your pallas kernel here")


# speedup vs baseline: 1.1447x; 1.1447x over previous
import functools

import jax
import jax.numpy as jnp
from jax import lax
from jax.experimental import pallas as pl
from jax.experimental.pallas import tpu as pltpu
from jax.experimental.pallas import tpu_sc as plsc

B, V, D, NF, NNEG = 4096, 100000, 64, 3, 20
NI = 1 + NNEG
H1, H2, H3 = 256, 128, 64
TEMPERATURE = 0.02
EPS = 1e-8

NC, NS = 2, 16
NW = NC * NS

U_TOT = B * NF
I_TOT = B * NI
U_PER_W = U_TOT // NW
I_PER_W = I_TOT // NW
U_CHUNK = 384
I_CHUNK = 896


def _sc_gather(table, idx, per_w, chunk):
  total = idx.shape[0]
  n_chunks = per_w // chunk
  mesh = plsc.VectorSubcoreMesh(core_axis_name="c", subcore_axis_name="s")

  @functools.partial(
      pl.kernel,
      mesh=mesh,
      out_type=jax.ShapeDtypeStruct((total, D), jnp.float32),
      compiler_params=pltpu.CompilerParams(use_tc_tiling_on_sc=False),
      scratch_types=[
          pltpu.VMEM((chunk,), jnp.int32),
          pltpu.VMEM((chunk, D), jnp.float32),
          pltpu.SemaphoreType.DMA,
      ],
  )
  def k(table_hbm, idx_hbm, out_hbm, idx_v, rows_v, sem):
    wid = lax.axis_index("s") * NC + lax.axis_index("c")
    base = wid * per_w

    @pl.loop(0, n_chunks)
    def _(ci):
      off = base + ci * chunk
      pltpu.sync_copy(idx_hbm.at[pl.ds(off, chunk)], idx_v)
      pltpu.async_copy(table_hbm.at[idx_v], rows_v, sem).wait()
      pltpu.sync_copy(rows_v, out_hbm.at[pl.ds(off, chunk)])

  return k(table, idx)


MLP_BLK = 1024


def _mlp_body(u_ref, w1_ref, b1_ref, w2_ref, b2_ref, w3_ref, b3_ref, o_ref):
  h = jnp.dot(u_ref[...], w1_ref[...], preferred_element_type=jnp.float32)
  h = jnp.maximum(h + b1_ref[...], 0.0)
  h = jnp.dot(h, w2_ref[...], preferred_element_type=jnp.float32)
  h = jnp.maximum(h + b2_ref[...], 0.0)
  h = jnp.dot(h, w3_ref[...], preferred_element_type=jnp.float32)
  o_ref[...] = jnp.maximum(h + b3_ref[...], 0.0)


def _mlp(u, W1, b1, W2, b2, W3, b3):
  full = lambda shape: pl.BlockSpec(shape, lambda i: (0,) * len(shape))
  return pl.pallas_call(
      _mlp_body,
      grid=(B // MLP_BLK,),
      in_specs=[
          pl.BlockSpec((MLP_BLK, NF * D), lambda i: (i, 0)),
          full((NF * D, H1)), full((1, H1)),
          full((H1, H2)), full((1, H2)),
          full((H2, H3)), full((1, H3)),
      ],
      out_specs=pl.BlockSpec((MLP_BLK, H3), lambda i: (i, 0)),
      out_shape=jax.ShapeDtypeStruct((B, H3), jnp.float32),
  )(u, W1, b1.reshape(1, H1), W2, b2.reshape(1, H2), W3, b3.reshape(1, H3))


COS_BLK = 1024


def _cosine_body(u_ref, it_ref, o_ref):
  u = u_ref[...]
  un = jnp.sqrt(jnp.sum(u * u, axis=-1, keepdims=True))
  cols = []
  for k in range(NI):
    itk = it_ref[k]
    dot = jnp.sum(u * itk, axis=-1, keepdims=True)
    inorm = jnp.sqrt(jnp.sum(itk * itk, axis=-1, keepdims=True))
    cols.append(dot / jnp.maximum(un * inorm, EPS))
  o_ref[...] = jnp.concatenate(cols, axis=1) * (1.0 / TEMPERATURE)


def _cosine(user_emb, item_rows):
  return pl.pallas_call(
      _cosine_body,
      grid=(B // COS_BLK,),
      in_specs=[
          pl.BlockSpec((COS_BLK, D), lambda i: (i, 0)),
          pl.BlockSpec((NI, COS_BLK, D), lambda i: (0, i, 0)),
      ],
      out_specs=pl.BlockSpec((COS_BLK, NI), lambda i: (i, 0)),
      out_shape=jax.ShapeDtypeStruct((B, NI), jnp.float32),
  )(user_emb, item_rows)


def kernel(user_idx, pos_item_idx, neg_item_idx, user_tables, item_table,
           W1, b1, W2, b2, W3, b3):
  user_flat = user_tables.reshape(NF * V, D)
  uidx = (user_idx.astype(jnp.int32)
          + (jnp.arange(NF, dtype=jnp.int32) * V)[None, :]).reshape(-1)
  iidx = jnp.concatenate(
      [pos_item_idx.astype(jnp.int32)[:, None],
       neg_item_idx.astype(jnp.int32)], axis=1).T.reshape(-1)

  u_rows = _sc_gather(user_flat, uidx, U_PER_W, U_CHUNK)
  it_rows = _sc_gather(item_table, iidx, I_PER_W, I_CHUNK)

  user_emb = _mlp(u_rows.reshape(B, NF * D), W1, b1, W2, b2, W3, b3)
  return _cosine(user_emb, it_rows.reshape(NI, B, D))
